# baseline (speedup 1.0000x reference)
import functools

import jax
import jax.numpy as jnp
import numpy as np
from jax import lax
from jax.experimental import pallas as pl
from jax.experimental.pallas import tpu as pltpu
from jax.experimental.pallas import tpu_sc as plsc


def _make_sc_gather(B, DEG, D, N):
    info = plsc.get_sparse_core_info()
    NW = info.num_cores * info.num_subcores
    b_per_w = B // NW
    CH = 4
    ROWS = CH * DEG
    n_chunks = b_per_w // CH
    n_pairs = n_chunks // 2
    DW = D // 2

    mesh = plsc.VectorSubcoreMesh(core_axis_name="c", subcore_axis_name="s")

    @functools.partial(
        pl.kernel,
        mesh=mesh,
        out_type=[
            jax.ShapeDtypeStruct((B, D), jnp.float32),
            jax.ShapeDtypeStruct((B, D), jnp.float32),
        ],
        scratch_types=[
            pltpu.VMEM((b_per_w * DEG,), jnp.int32),
            pltpu.VMEM((b_per_w,), jnp.int32),
            pltpu.VMEM((ROWS, DW), jnp.int32),
            pltpu.VMEM((ROWS, DW), jnp.int32),
            pltpu.VMEM((b_per_w, D), jnp.float32),
            pltpu.VMEM((b_per_w, D), jnp.float32),
            pltpu.SemaphoreType.DMA,
            pltpu.SemaphoreType.DMA,
            pltpu.SemaphoreType.DMA,
        ],
    )
    def sc_gather(tab_f32_hbm, tab_i32_hbm, nodes_hbm, neigh_hbm,
                  self_out, sum_out,
                  nidx, sidx, buf0, buf1, sumbuf, selfbuf,
                  sem0, sem1, sem2):
        wid = lax.axis_index("s") * info.num_cores + lax.axis_index("c")
        base = wid * b_per_w

        pltpu.sync_copy(neigh_hbm.at[pl.ds(base * DEG, b_per_w * DEG)], nidx)
        pltpu.sync_copy(nodes_hbm.at[pl.ds(base, b_per_w)], sidx)

        self_cp = pltpu.make_async_copy(tab_f32_hbm.at[sidx], selfbuf, sem2)
        self_cp.start()

        bufs = (buf0, buf1)
        sems = (sem0, sem1)

        def gather(c, slot):
            pltpu.make_async_copy(
                tab_i32_hbm.at[nidx.at[pl.ds(c * ROWS, ROWS)]],
                bufs[slot], sems[slot]).start()

        def gwait(slot):
            pltpu.make_async_copy(
                tab_i32_hbm.at[nidx.at[pl.ds(0, ROWS)]],
                bufs[slot], sems[slot]).wait()

        hi_mask = jnp.full((16,), -65536, jnp.int32)

        def reduce(c, slot):
            buf = bufs[slot]

            def body(i, carry):
                row = c * CH + i
                for g in range(DW // 16):
                    col = g * 16
                    x = buf[i * DEG, col:col + 16]
                    acc_e = lax.bitcast_convert_type(lax.shift_left(x, 16), jnp.float32)
                    acc_o = lax.bitcast_convert_type(x & hi_mask, jnp.float32)
                    for j in range(1, DEG):
                        xj = buf[i * DEG + j, col:col + 16]
                        acc_e = acc_e + lax.bitcast_convert_type(
                            lax.shift_left(xj, 16), jnp.float32)
                        acc_o = acc_o + lax.bitcast_convert_type(xj & hi_mask, jnp.float32)
                    sumbuf[row, 2 * col:2 * col + 16] = acc_e
                    sumbuf[row, 2 * col + 16:2 * col + 32] = acc_o
                return carry

            lax.fori_loop(0, CH, body, 0)

        gather(0, 0)

        def pair(gg, carry):
            c0 = gg * 2
            gather(c0 + 1, 1)
            gwait(0)
            reduce(c0, 0)

            @pl.when(gg < n_pairs - 1)
            def _():
                gather(c0 + 2, 0)

            gwait(1)
            reduce(c0 + 1, 1)
            return carry

        lax.fori_loop(0, n_pairs, pair, 0)

        sum_cp = pltpu.make_async_copy(
            sumbuf, sum_out.at[pl.ds(base, b_per_w)], sem0)
        sum_cp.start()
        self_cp.wait()
        pltpu.sync_copy(selfbuf, self_out.at[pl.ds(base, b_per_w)])
        sum_cp.wait()

    return sc_gather


def _make_tc_mlp(B, D, E, DEG):
    BM = 512
    inv = 1.0 / DEG

    def body(x1_ref, x2_ref, w_ref, b_ref, o_ref):
        xc = jnp.concatenate([x1_ref[...], x2_ref[...] * inv], axis=1)
        acc = jnp.dot(xc, w_ref[...], preferred_element_type=jnp.float32)
        o_ref[...] = jnp.maximum(acc + b_ref[...], 0.0)

    return pl.pallas_call(
        body,
        grid=(B // BM,),
        in_specs=[
            pl.BlockSpec((BM, D), lambda i: (i, 0)),
            pl.BlockSpec((BM, D), lambda i: (i, 0)),
            pl.BlockSpec((2 * D, E), lambda i: (0, 0)),
            pl.BlockSpec((1, E), lambda i: (0, 0)),
        ],
        out_specs=pl.BlockSpec((BM, E), lambda i: (i, 0)),
        out_shape=jax.ShapeDtypeStruct((B, E), jnp.float32),
    )


def kernel(features_table, W, b, nodes, neighbors):
    N, D = features_table.shape
    B, DEG = neighbors.shape
    E = W.shape[1]

    table_i32 = lax.bitcast_convert_type(
        features_table.astype(jnp.bfloat16).reshape(N, D // 2, 2),
        jnp.int32)

    sc_gather = _make_sc_gather(B, DEG, D, N)
    self_feats, neigh_sum_perm = sc_gather(
        features_table, table_i32, nodes, neighbors.reshape(-1))

    q = np.arange(D)
    g, r = q // 32, q % 32
    src = g * 32 + np.where(r < 16, 2 * r, 2 * (r - 16) + 1)
    W_perm = jnp.concatenate([W[:D], W[D:][src]], axis=0)

    tc_mlp = _make_tc_mlp(B, D, E, DEG)
    return tc_mlp(self_feats, neigh_sum_perm, W_perm, b.reshape(1, E))


# speedup vs baseline: 4.4846x; 4.4846x over previous
import functools

import jax
import jax.numpy as jnp
import numpy as np
from jax import lax
from jax.experimental import pallas as pl
from jax.experimental.pallas import tpu as pltpu
from jax.experimental.pallas import tpu_sc as plsc


def _make_sc_gather(B, DEG, D, N):
    info = plsc.get_sparse_core_info()
    NW = info.num_cores * info.num_subcores
    b_per_w = B // NW
    CH = 4
    ROWS = CH * DEG
    n_chunks = b_per_w // CH
    n_pairs = n_chunks // 2
    DW = D // 2

    mesh = plsc.VectorSubcoreMesh(core_axis_name="c", subcore_axis_name="s")

    @functools.partial(
        pl.kernel,
        mesh=mesh,
        out_type=[
            jax.ShapeDtypeStruct((B, D), jnp.float32),
            jax.ShapeDtypeStruct((B, D), jnp.float32),
        ],
        scratch_types=[
            pltpu.VMEM((b_per_w * DEG,), jnp.int32),
            pltpu.VMEM((b_per_w,), jnp.int32),
            pltpu.VMEM((ROWS, DW), jnp.int32),
            pltpu.VMEM((ROWS, DW), jnp.int32),
            pltpu.VMEM((b_per_w, D), jnp.float32),
            pltpu.VMEM((b_per_w, D), jnp.float32),
            pltpu.SemaphoreType.DMA,
            pltpu.SemaphoreType.DMA,
            pltpu.SemaphoreType.DMA,
        ],
    )
    def sc_gather(tab_f32_hbm, tab_i32_hbm, nodes_hbm, neigh_hbm,
                  self_out, sum_out,
                  nidx, sidx, buf0, buf1, sumbuf, selfbuf,
                  sem0, sem1, sem2):
        wid = lax.axis_index("s") * info.num_cores + lax.axis_index("c")
        base = wid * b_per_w

        pltpu.sync_copy(neigh_hbm.at[pl.ds(base * DEG, b_per_w * DEG)], nidx)
        pltpu.sync_copy(nodes_hbm.at[pl.ds(base, b_per_w)], sidx)

        self_cp = pltpu.make_async_copy(tab_f32_hbm.at[sidx], selfbuf, sem2)
        self_cp.start()

        bufs = (buf0, buf1)
        sems = (sem0, sem1)

        def gather(c, slot):
            pltpu.make_async_copy(
                tab_i32_hbm.at[nidx.at[pl.ds(c * ROWS, ROWS)]],
                bufs[slot], sems[slot]).start()

        def gwait(slot):
            pltpu.make_async_copy(
                tab_i32_hbm.at[nidx.at[pl.ds(0, ROWS)]],
                bufs[slot], sems[slot]).wait()

        hi_mask = jnp.full((16,), -65536, jnp.int32)

        def reduce(c, slot):
            buf = bufs[slot]

            def body(i, carry):
                row = c * CH + i
                for g in range(DW // 16):
                    col = g * 16
                    x = buf[i * DEG, col:col + 16]
                    acc_e = lax.bitcast_convert_type(lax.shift_left(x, 16), jnp.float32)
                    acc_o = lax.bitcast_convert_type(x & hi_mask, jnp.float32)
                    for j in range(1, DEG):
                        xj = buf[i * DEG + j, col:col + 16]
                        acc_e = acc_e + lax.bitcast_convert_type(
                            lax.shift_left(xj, 16), jnp.float32)
                        acc_o = acc_o + lax.bitcast_convert_type(xj & hi_mask, jnp.float32)
                    sumbuf[row, col:col + 16] = acc_e
                    sumbuf[row, DW + col:DW + col + 16] = acc_o
                return carry

            lax.fori_loop(0, CH, body, 0)

        gather(0, 0)

        def pair(gg, carry):
            c0 = gg * 2
            gather(c0 + 1, 1)
            gwait(0)
            reduce(c0, 0)

            @pl.when(gg < n_pairs - 1)
            def _():
                gather(c0 + 2, 0)

            gwait(1)
            reduce(c0 + 1, 1)
            return carry

        lax.fori_loop(0, n_pairs, pair, 0)

        sum_cp = pltpu.make_async_copy(
            sumbuf, sum_out.at[pl.ds(base, b_per_w)], sem0)
        sum_cp.start()
        self_cp.wait()
        pltpu.sync_copy(selfbuf, self_out.at[pl.ds(base, b_per_w)])
        sum_cp.wait()

    return sc_gather


def _make_tc_pack(N, D):
    BM = 1000
    assert N % BM == 0

    def body(x_ref, o_ref):
        x = x_ref[...]
        lo = lax.bitcast_convert_type(x[:, :D // 2], jnp.int32)
        hi = lax.bitcast_convert_type(x[:, D // 2:], jnp.int32)

        def rne(bits):
            return bits + 0x7FFF + (lax.shift_right_logical(bits, 16) & 1)

        o_ref[...] = (
            lax.shift_right_logical(rne(lo), 16)
            | (rne(hi) & jnp.int32(-65536)))

    return pl.pallas_call(
        body,
        grid=(N // BM,),
        in_specs=[pl.BlockSpec((BM, D), lambda i: (i, 0))],
        out_specs=pl.BlockSpec((BM, D // 2), lambda i: (i, 0)),
        out_shape=jax.ShapeDtypeStruct((N, D // 2), jnp.int32),
    )


def _make_tc_mlp(B, D, E, DEG):
    BM = 512
    inv = 1.0 / DEG

    def body(x1_ref, x2_ref, w_ref, b_ref, o_ref):
        xc = jnp.concatenate([x1_ref[...], x2_ref[...] * inv], axis=1)
        acc = jnp.dot(xc, w_ref[...], preferred_element_type=jnp.float32)
        o_ref[...] = jnp.maximum(acc + b_ref[...], 0.0)

    return pl.pallas_call(
        body,
        grid=(B // BM,),
        in_specs=[
            pl.BlockSpec((BM, D), lambda i: (i, 0)),
            pl.BlockSpec((BM, D), lambda i: (i, 0)),
            pl.BlockSpec((2 * D, E), lambda i: (0, 0)),
            pl.BlockSpec((1, E), lambda i: (0, 0)),
        ],
        out_specs=pl.BlockSpec((BM, E), lambda i: (i, 0)),
        out_shape=jax.ShapeDtypeStruct((B, E), jnp.float32),
    )


def kernel(features_table, W, b, nodes, neighbors):
    N, D = features_table.shape
    B, DEG = neighbors.shape
    E = W.shape[1]

    table_i32 = _make_tc_pack(N, D)(features_table)

    sc_gather = _make_sc_gather(B, DEG, D, N)
    self_feats, neigh_sum_perm = sc_gather(
        features_table, table_i32, nodes, neighbors.reshape(-1))

    tc_mlp = _make_tc_mlp(B, D, E, DEG)
    return tc_mlp(self_feats, neigh_sum_perm, W, b.reshape(1, E))
